# Initial kernel scaffold; baseline (speedup 1.0000x reference)
import functools

import jax
import jax.numpy as jnp
from jax import lax
from jax.experimental import pallas as pl
from jax.experimental.pallas import tpu as pltpu
from jax.experimental.pallas import tpu_sc as plsc

BN_EPS = 1e-5
_C = 80


def _seg_sum_partials(h, src2, dst2):
    n, d = h.shape
    r, c = src2.shape
    info = plsc.get_sparse_core_info()
    nc, ns = info.num_cores, info.num_subcores
    nw = nc * ns
    rows_w = r // nw
    acc_rows = n // ns
    mesh = plsc.VectorSubcoreMesh(core_axis_name="c", subcore_axis_name="s")

    @functools.partial(
        pl.kernel,
        out_type=jax.ShapeDtypeStruct((nc, n, d), jnp.float32),
        mesh=mesh,
        scratch_types=[
            pltpu.VMEM((rows_w, c), jnp.int32),
            pltpu.VMEM((rows_w, c), jnp.int32),
            pltpu.VMEM((c, d), jnp.float32),
            pltpu.VMEM_SHARED((n, d), jnp.float32),
            pltpu.SemaphoreType.DMA,
        ],
    )
    def seg_sum(h_hbm, src_hbm, dst_hbm, out_hbm, src_v, dst_v, rows0, acc, sem0):
        cid = lax.axis_index("c")
        sid = lax.axis_index("s")
        wid = sid * nc + cid
        row0 = wid * rows_w

        pltpu.sync_copy(src_hbm.at[pl.ds(row0, rows_w)], src_v)
        pltpu.sync_copy(dst_hbm.at[pl.ds(row0, rows_w)], dst_v)

        def zbody(i, carry):
            for j in range(d // 16):
                rows0[i, pl.ds(j * 16, 16)] = jnp.zeros((16,), jnp.float32)
            return carry

        lax.fori_loop(0, c, zbody, 0)
        base = sid * acc_rows
        full, rem = acc_rows // c, acc_rows % c
        for k in range(full):
            pltpu.sync_copy(rows0, acc.at[pl.ds(base + k * c, c)])
        if rem:
            pltpu.sync_copy(rows0.at[pl.ds(0, rem)],
                            acc.at[pl.ds(base + full * c, rem)])
        plsc.subcore_barrier()

        def body(j, carry):
            pltpu.async_copy(h_hbm.at[src_v.at[j]], rows0, sem0).wait()
            pltpu.sync_copy(rows0, acc.at[dst_v.at[j]], add=True)
            return carry

        lax.fori_loop(0, rows_w, body, 0)

        plsc.subcore_barrier()
        pltpu.sync_copy(acc.at[pl.ds(base, acc_rows)],
                        out_hbm.at[cid, pl.ds(base, acc_rows)])

    return seg_sum(h, src2, dst2)


def _init_matmul(x, w0):
    def body(x_ref, w_ref, o_ref):
        o_ref[...] = lax.dot_general(
            x_ref[...], w_ref[...], (((1,), (1,)), ((), ())),
            preferred_element_type=jnp.float32)

    return pl.pallas_call(
        body,
        out_shape=jax.ShapeDtypeStruct((x.shape[0], w0.shape[0]), jnp.float32),
    )(x, w0)


def _gin_mlp(h, parts, eps, w1, b1, g, be, w2, b2):
    n, d = h.shape

    def body(h_ref, p_ref, eps_ref, w1_ref, b1_ref, g_ref, be_ref, w2_ref,
             b2_ref, o_ref):
        agg = p_ref[0] + p_ref[1]
        t = (1.0 + eps_ref[0, 0]) * h_ref[...] + agg
        u = lax.dot_general(t, w1_ref[...], (((1,), (1,)), ((), ())),
                            preferred_element_type=jnp.float32) + b1_ref[...]
        mean = jnp.mean(u, axis=0, keepdims=True)
        var = jnp.mean(jnp.square(u - mean), axis=0, keepdims=True)
        un = (u - mean) * lax.rsqrt(var + BN_EPS) * g_ref[...] + be_ref[...]
        un = jnp.maximum(un, 0.0)
        v = lax.dot_general(un, w2_ref[...], (((1,), (1,)), ((), ())),
                            preferred_element_type=jnp.float32) + b2_ref[...]
        o_ref[...] = jnp.maximum(v, 0.0)

    return pl.pallas_call(
        body,
        in_specs=[
            pl.BlockSpec(memory_space=pltpu.VMEM),
            pl.BlockSpec(memory_space=pltpu.VMEM),
            pl.BlockSpec(memory_space=pltpu.SMEM),
            pl.BlockSpec(memory_space=pltpu.VMEM),
            pl.BlockSpec(memory_space=pltpu.VMEM),
            pl.BlockSpec(memory_space=pltpu.VMEM),
            pl.BlockSpec(memory_space=pltpu.VMEM),
            pl.BlockSpec(memory_space=pltpu.VMEM),
            pl.BlockSpec(memory_space=pltpu.VMEM),
        ],
        out_shape=jax.ShapeDtypeStruct((n, d), jnp.float32),
    )(h, parts, eps.reshape(1, 1), w1, b1.reshape(1, d), g.reshape(1, d),
      be.reshape(1, d), w2, b2.reshape(1, d))


def kernel(x, edge_index, W0, eps0, W1_0, b1_0, g_0, be_0, W2_0, b2_0,
           eps1, W1_1, b1_1, g_1, be_1, W2_1, b2_1):
    n = x.shape[0]
    d = W0.shape[0]
    e = edge_index.shape[1]
    src2 = edge_index[0].reshape(e // _C, _C)
    dst2 = edge_index[1].reshape(e // _C, _C)

    h0 = _init_matmul(x, W0)
    p0 = _seg_sum_partials(h0, src2, dst2)
    h1 = _gin_mlp(h0, p0, eps0, W1_0, b1_0, g_0, be_0, W2_0, b2_0)
    p1 = _seg_sum_partials(h1, src2, dst2)
    h2 = _gin_mlp(h1, p1, eps1, W1_1, b1_1, g_1, be_1, W2_1, b2_1)
    return h2.reshape(1, n, d)


# speedup vs baseline: 7.7109x; 7.7109x over previous
import functools

import jax
import jax.numpy as jnp
from jax import lax
from jax.experimental import pallas as pl
from jax.experimental.pallas import tpu as pltpu
from jax.experimental.pallas import tpu_sc as plsc

BN_EPS = 1e-5
_C = 125


def _seg_sum_partials(h, src2, dst2):
    n, d = h.shape
    r, c = src2.shape
    info = plsc.get_sparse_core_info()
    nc, ns = info.num_cores, info.num_subcores
    nw = nc * ns
    rows_w = r // nw
    zc = 128
    n_pad = ((n + ns * zc - 1) // (ns * zc)) * ns * zc
    acc_rows = n_pad // ns
    mesh = plsc.VectorSubcoreMesh(core_axis_name="c", subcore_axis_name="s")

    @functools.partial(
        pl.kernel,
        out_type=jax.ShapeDtypeStruct((nc, n, d), jnp.float32),
        mesh=mesh,
        scratch_types=[
            pltpu.VMEM((rows_w, c), jnp.int32),
            pltpu.VMEM((rows_w, c), jnp.int32),
            pltpu.VMEM((zc, d), jnp.float32),
            pltpu.VMEM_SHARED((n_pad, d), jnp.float32),
            pltpu.SemaphoreType.DMA,
        ],
    )
    def seg_sum(h_hbm, src_hbm, dst_hbm, out_hbm, src_v, dst_v, rows0, acc, sem0):
        cid = lax.axis_index("c")
        sid = lax.axis_index("s")
        wid = sid * nc + cid
        row0 = wid * rows_w

        pltpu.sync_copy(src_hbm.at[pl.ds(row0, rows_w)], src_v)
        pltpu.sync_copy(dst_hbm.at[pl.ds(row0, rows_w)], dst_v)

        def zbody(i, carry):
            for j in range(d // 16):
                rows0[i, pl.ds(j * 16, 16)] = jnp.zeros((16,), jnp.float32)
            return carry

        lax.fori_loop(0, zc, zbody, 0)
        base = sid * acc_rows
        for k in range(acc_rows // zc):
            pltpu.sync_copy(rows0, acc.at[pl.ds(base + k * zc, zc)])
        plsc.subcore_barrier()

        def body(j, carry):
            pltpu.async_copy(h_hbm.at[src_v.at[j]], rows0.at[pl.ds(0, c)],
                             sem0).wait()
            pltpu.sync_copy(rows0.at[pl.ds(0, c)], acc.at[dst_v.at[j]],
                            add=True)
            return carry

        lax.fori_loop(0, rows_w, body, 0)

        plsc.subcore_barrier()
        tail = n - (ns - 1) * acc_rows

        @pl.when(sid < ns - 1)
        def _():
            pltpu.sync_copy(acc.at[pl.ds(base, acc_rows)],
                            out_hbm.at[cid, pl.ds(base, acc_rows)])

        @pl.when(sid == ns - 1)
        def _():
            pltpu.sync_copy(acc.at[pl.ds(base, tail)],
                            out_hbm.at[cid, pl.ds(base, tail)])

    return seg_sum(h, src2, dst2)


def _init_matmul(x, w0):
    def body(x_ref, w_ref, o_ref):
        o_ref[...] = lax.dot_general(
            x_ref[...], w_ref[...], (((1,), (1,)), ((), ())),
            preferred_element_type=jnp.float32)

    return pl.pallas_call(
        body,
        out_shape=jax.ShapeDtypeStruct((x.shape[0], w0.shape[0]), jnp.float32),
    )(x, w0)


def _gin_mlp(h, parts, eps, w1, b1, g, be, w2, b2):
    n, d = h.shape

    def body(h_ref, p_ref, eps_ref, w1_ref, b1_ref, g_ref, be_ref, w2_ref,
             b2_ref, o_ref):
        agg = p_ref[0] + p_ref[1]
        t = (1.0 + eps_ref[0, 0]) * h_ref[...] + agg
        u = lax.dot_general(t, w1_ref[...], (((1,), (1,)), ((), ())),
                            preferred_element_type=jnp.float32) + b1_ref[...]
        mean = jnp.mean(u, axis=0, keepdims=True)
        var = jnp.mean(jnp.square(u - mean), axis=0, keepdims=True)
        un = (u - mean) * lax.rsqrt(var + BN_EPS) * g_ref[...] + be_ref[...]
        un = jnp.maximum(un, 0.0)
        v = lax.dot_general(un, w2_ref[...], (((1,), (1,)), ((), ())),
                            preferred_element_type=jnp.float32) + b2_ref[...]
        o_ref[...] = jnp.maximum(v, 0.0)

    return pl.pallas_call(
        body,
        in_specs=[
            pl.BlockSpec(memory_space=pltpu.VMEM),
            pl.BlockSpec(memory_space=pltpu.VMEM),
            pl.BlockSpec(memory_space=pltpu.SMEM),
            pl.BlockSpec(memory_space=pltpu.VMEM),
            pl.BlockSpec(memory_space=pltpu.VMEM),
            pl.BlockSpec(memory_space=pltpu.VMEM),
            pl.BlockSpec(memory_space=pltpu.VMEM),
            pl.BlockSpec(memory_space=pltpu.VMEM),
            pl.BlockSpec(memory_space=pltpu.VMEM),
        ],
        out_shape=jax.ShapeDtypeStruct((n, d), jnp.float32),
    )(h, parts, eps.reshape(1, 1), w1, b1.reshape(1, d), g.reshape(1, d),
      be.reshape(1, d), w2, b2.reshape(1, d))


def kernel(x, edge_index, W0, eps0, W1_0, b1_0, g_0, be_0, W2_0, b2_0,
           eps1, W1_1, b1_1, g_1, be_1, W2_1, b2_1):
    n = x.shape[0]
    d = W0.shape[0]
    e = edge_index.shape[1]
    src2 = edge_index[0].reshape(e // _C, _C)
    dst2 = edge_index[1].reshape(e // _C, _C)

    h0 = _init_matmul(x, W0)
    p0 = _seg_sum_partials(h0, src2, dst2)
    h1 = _gin_mlp(h0, p0, eps0, W1_0, b1_0, g_0, be_0, W2_0, b2_0)
    p1 = _seg_sum_partials(h1, src2, dst2)
    h2 = _gin_mlp(h1, p1, eps1, W1_1, b1_1, g_1, be_1, W2_1, b2_1)
    return h2.reshape(1, n, d)


# speedup vs baseline: 9.9762x; 1.2938x over previous
import functools

import jax
import jax.numpy as jnp
from jax import lax
from jax.experimental import pallas as pl
from jax.experimental.pallas import tpu as pltpu
from jax.experimental.pallas import tpu_sc as plsc

BN_EPS = 1e-5
_C = 125


def _seg_sum_partials(h, src2, dst2):
    n, d = h.shape
    r, c = src2.shape
    info = plsc.get_sparse_core_info()
    nc, ns = info.num_cores, info.num_subcores
    nw = nc * ns
    rows_w = r // nw
    grp = 8
    n_grp = rows_w // grp
    n_pad = ((n + ns * 8 - 1) // (ns * 8)) * ns * 8
    acc_rows = n_pad // ns
    mesh = plsc.VectorSubcoreMesh(core_axis_name="c", subcore_axis_name="s")

    @functools.partial(
        pl.kernel,
        out_type=jax.ShapeDtypeStruct((nc, n, d), jnp.float32),
        mesh=mesh,
        scratch_types=[
            [pltpu.VMEM((grp, c), jnp.int32)] * 2,
            [pltpu.VMEM((grp, c), jnp.int32)] * 2,
            [pltpu.VMEM((c, d), jnp.float32)] * 2,
            pltpu.VMEM_SHARED((n_pad, d), jnp.float32),
            [pltpu.SemaphoreType.DMA] * 2,
            [pltpu.SemaphoreType.DMA] * 2,
            pltpu.SemaphoreType.DMA,
        ],
    )
    def seg_sum(h_hbm, src_hbm, dst_hbm, out_hbm, srcg, dstg, rows, acc,
                semg, sems, semi):
        cid = lax.axis_index("c")
        sid = lax.axis_index("s")
        wid = sid * nc + cid
        row0 = wid * rows_w

        def zbody(i, carry):
            for j in range(d // 16):
                rows[0][i, pl.ds(j * 16, 16)] = jnp.zeros((16,), jnp.float32)
            return carry

        lax.fori_loop(0, c, zbody, 0)
        base = sid * acc_rows
        zc = (c // 8) * 8
        nfull, rem = acc_rows // zc, acc_rows % zc
        for k in range(nfull):
            pltpu.sync_copy(rows[0].at[pl.ds(0, zc)],
                            acc.at[pl.ds(base + k * zc, zc)])
        if rem:
            pltpu.sync_copy(rows[0].at[pl.ds(0, rem)],
                            acc.at[pl.ds(base + nfull * zc, rem)])
        plsc.subcore_barrier()

        def start_gather(sref, k, b):
            pltpu.async_copy(h_hbm.at[sref.at[k]], rows[b], semg[b])

        def wait_gather(b):
            pltpu.make_async_copy(h_hbm.at[srcg[0].at[0]], rows[b],
                                  semg[b]).wait()

        def start_scatter(dref, k, b):
            pltpu.async_copy(rows[b], acc.at[dref.at[k]], sems[b], add=True)

        def wait_scatter(b):
            pltpu.make_async_copy(rows[b], acc.at[dstg[0].at[0]],
                                  sems[b]).wait()

        def prefetch_idx(g, sdst, ddst):
            pltpu.async_copy(src_hbm.at[pl.ds(row0 + g * grp, grp)], sdst,
                             semi)
            pltpu.async_copy(dst_hbm.at[pl.ds(row0 + g * grp, grp)], ddst,
                             semi)

        def wait_idx():
            pltpu.make_async_copy(src_hbm.at[pl.ds(0, grp)], srcg[0],
                                  semi).wait()
            pltpu.make_async_copy(src_hbm.at[pl.ds(0, grp)], dstg[0],
                                  semi).wait()

        def do_group(g, cs, cd, nxs, nxd):
            @pl.when(g + 1 < n_grp)
            def _():
                prefetch_idx(g + 1, nxs, nxd)

            for k in range(grp):
                b = k % 2
                nb = 1 - b
                j = g * grp + k
                wait_gather(b)

                @pl.when(j >= 1)
                def _():
                    wait_scatter(nb)

                if k == grp - 1:
                    @pl.when(g + 1 < n_grp)
                    def _():
                        wait_idx()
                        start_gather(nxs, 0, nb)
                else:
                    start_gather(cs, k + 1, nb)
                start_scatter(cd, k, b)

        pltpu.sync_copy(src_hbm.at[pl.ds(row0, grp)], srcg[0])
        pltpu.sync_copy(dst_hbm.at[pl.ds(row0, grp)], dstg[0])
        start_gather(srcg[0], 0, 0)

        def body(g, carry):
            even = lax.rem(g, 2) == 0

            @pl.when(even)
            def _():
                do_group(g, srcg[0], dstg[0], srcg[1], dstg[1])

            @pl.when(jnp.logical_not(even))
            def _():
                do_group(g, srcg[1], dstg[1], srcg[0], dstg[0])

            return carry

        lax.fori_loop(0, n_grp, body, 0)
        wait_scatter((rows_w - 1) % 2)

        plsc.subcore_barrier()
        tail = n - (ns - 1) * acc_rows

        @pl.when(sid < ns - 1)
        def _():
            pltpu.sync_copy(acc.at[pl.ds(base, acc_rows)],
                            out_hbm.at[cid, pl.ds(base, acc_rows)])

        @pl.when(sid == ns - 1)
        def _():
            pltpu.sync_copy(acc.at[pl.ds(base, tail)],
                            out_hbm.at[cid, pl.ds(base, tail)])

    return seg_sum(h, src2, dst2)


def _init_matmul(x, w0):
    def body(x_ref, w_ref, o_ref):
        o_ref[...] = lax.dot_general(
            x_ref[...], w_ref[...], (((1,), (1,)), ((), ())),
            preferred_element_type=jnp.float32)

    return pl.pallas_call(
        body,
        out_shape=jax.ShapeDtypeStruct((x.shape[0], w0.shape[0]), jnp.float32),
    )(x, w0)


def _gin_mlp(h, parts, eps, w1, b1, g, be, w2, b2):
    n, d = h.shape

    def body(h_ref, p_ref, eps_ref, w1_ref, b1_ref, g_ref, be_ref, w2_ref,
             b2_ref, o_ref):
        agg = p_ref[0] + p_ref[1]
        t = (1.0 + eps_ref[0, 0]) * h_ref[...] + agg
        u = lax.dot_general(t, w1_ref[...], (((1,), (1,)), ((), ())),
                            preferred_element_type=jnp.float32) + b1_ref[...]
        mean = jnp.mean(u, axis=0, keepdims=True)
        var = jnp.mean(jnp.square(u - mean), axis=0, keepdims=True)
        un = (u - mean) * lax.rsqrt(var + BN_EPS) * g_ref[...] + be_ref[...]
        un = jnp.maximum(un, 0.0)
        v = lax.dot_general(un, w2_ref[...], (((1,), (1,)), ((), ())),
                            preferred_element_type=jnp.float32) + b2_ref[...]
        o_ref[...] = jnp.maximum(v, 0.0)

    return pl.pallas_call(
        body,
        in_specs=[
            pl.BlockSpec(memory_space=pltpu.VMEM),
            pl.BlockSpec(memory_space=pltpu.VMEM),
            pl.BlockSpec(memory_space=pltpu.SMEM),
            pl.BlockSpec(memory_space=pltpu.VMEM),
            pl.BlockSpec(memory_space=pltpu.VMEM),
            pl.BlockSpec(memory_space=pltpu.VMEM),
            pl.BlockSpec(memory_space=pltpu.VMEM),
            pl.BlockSpec(memory_space=pltpu.VMEM),
            pl.BlockSpec(memory_space=pltpu.VMEM),
        ],
        out_shape=jax.ShapeDtypeStruct((n, d), jnp.float32),
    )(h, parts, eps.reshape(1, 1), w1, b1.reshape(1, d), g.reshape(1, d),
      be.reshape(1, d), w2, b2.reshape(1, d))


def kernel(x, edge_index, W0, eps0, W1_0, b1_0, g_0, be_0, W2_0, b2_0,
           eps1, W1_1, b1_1, g_1, be_1, W2_1, b2_1):
    n = x.shape[0]
    d = W0.shape[0]
    e = edge_index.shape[1]
    src2 = edge_index[0].reshape(e // _C, _C)
    dst2 = edge_index[1].reshape(e // _C, _C)

    h0 = _init_matmul(x, W0)
    p0 = _seg_sum_partials(h0, src2, dst2)
    h1 = _gin_mlp(h0, p0, eps0, W1_0, b1_0, g_0, be_0, W2_0, b2_0)
    p1 = _seg_sum_partials(h1, src2, dst2)
    h2 = _gin_mlp(h1, p1, eps1, W1_1, b1_1, g_1, be_1, W2_1, b2_1)
    return h2.reshape(1, n, d)


# speedup vs baseline: 10.3127x; 1.0337x over previous
import functools

import jax
import jax.numpy as jnp
from jax import lax
from jax.experimental import pallas as pl
from jax.experimental.pallas import tpu as pltpu
from jax.experimental.pallas import tpu_sc as plsc

BN_EPS = 1e-5
_C = 125


def _seg_sum_partials(h, e2):
    n, d = h.shape
    r2, c = e2.shape
    r = r2 // 2
    info = plsc.get_sparse_core_info()
    nc, ns = info.num_cores, info.num_subcores
    nw = nc * ns
    rows_w = r // nw
    grp = 8
    n_grp = rows_w // grp
    n_pad = ((n + ns * 8 - 1) // (ns * 8)) * ns * 8
    acc_rows = n_pad // ns
    mesh = plsc.VectorSubcoreMesh(core_axis_name="c", subcore_axis_name="s")

    @functools.partial(
        pl.kernel,
        out_type=jax.ShapeDtypeStruct((nc, n, d), jnp.float32),
        mesh=mesh,
        scratch_types=[
            [pltpu.VMEM((grp, c), jnp.int32)] * 2,
            [pltpu.VMEM((grp, c), jnp.int32)] * 2,
            [pltpu.VMEM((c, d), jnp.float32)] * 2,
            pltpu.VMEM_SHARED((n_pad, d), jnp.float32),
            [pltpu.SemaphoreType.DMA] * 2,
            [pltpu.SemaphoreType.DMA] * 2,
            pltpu.SemaphoreType.DMA,
        ],
    )
    def seg_sum(h_hbm, e_hbm, out_hbm, srcg, dstg, rows, acc,
                semg, sems, semi):
        cid = lax.axis_index("c")
        sid = lax.axis_index("s")
        wid = sid * nc + cid
        row0 = wid * rows_w

        def zbody(i, carry):
            for j in range(d // 16):
                rows[0][i, pl.ds(j * 16, 16)] = jnp.zeros((16,), jnp.float32)
            return carry

        lax.fori_loop(0, c, zbody, 0)
        base = sid * acc_rows
        zc = (c // 8) * 8
        nfull, rem = acc_rows // zc, acc_rows % zc
        for k in range(nfull):
            pltpu.sync_copy(rows[0].at[pl.ds(0, zc)],
                            acc.at[pl.ds(base + k * zc, zc)])
        if rem:
            pltpu.sync_copy(rows[0].at[pl.ds(0, rem)],
                            acc.at[pl.ds(base + nfull * zc, rem)])
        plsc.subcore_barrier()

        def start_gather(sref, k, b):
            pltpu.async_copy(h_hbm.at[sref.at[k]], rows[b], semg[b])

        def wait_gather(b):
            pltpu.make_async_copy(h_hbm.at[srcg[0].at[0]], rows[b],
                                  semg[b]).wait()

        def start_scatter(dref, k, b):
            pltpu.async_copy(rows[b], acc.at[dref.at[k]], sems[b], add=True)

        def wait_scatter(b):
            pltpu.make_async_copy(rows[b], acc.at[dstg[0].at[0]],
                                  sems[b]).wait()

        def prefetch_idx(g, sdst, ddst):
            pltpu.async_copy(e_hbm.at[pl.ds(row0 + g * grp, grp)], sdst,
                             semi)
            pltpu.async_copy(e_hbm.at[pl.ds(r + row0 + g * grp, grp)], ddst,
                             semi)

        def wait_idx():
            pltpu.make_async_copy(e_hbm.at[pl.ds(0, grp)], srcg[0],
                                  semi).wait()
            pltpu.make_async_copy(e_hbm.at[pl.ds(0, grp)], dstg[0],
                                  semi).wait()

        def do_group(g, cs, cd, nxs, nxd):
            @pl.when(g + 1 < n_grp)
            def _():
                prefetch_idx(g + 1, nxs, nxd)

            for k in range(grp):
                b = k % 2
                nb = 1 - b
                j = g * grp + k
                wait_gather(b)

                @pl.when(j >= 1)
                def _():
                    wait_scatter(nb)

                if k == grp - 1:
                    @pl.when(g + 1 < n_grp)
                    def _():
                        wait_idx()
                        start_gather(nxs, 0, nb)
                else:
                    start_gather(cs, k + 1, nb)
                start_scatter(cd, k, b)

        pltpu.sync_copy(e_hbm.at[pl.ds(row0, grp)], srcg[0])
        pltpu.sync_copy(e_hbm.at[pl.ds(r + row0, grp)], dstg[0])
        start_gather(srcg[0], 0, 0)

        def body(g, carry):
            even = lax.rem(g, 2) == 0

            @pl.when(even)
            def _():
                do_group(g, srcg[0], dstg[0], srcg[1], dstg[1])

            @pl.when(jnp.logical_not(even))
            def _():
                do_group(g, srcg[1], dstg[1], srcg[0], dstg[0])

            return carry

        lax.fori_loop(0, n_grp, body, 0)
        wait_scatter((rows_w - 1) % 2)

        plsc.subcore_barrier()
        tail = n - (ns - 1) * acc_rows

        @pl.when(sid < ns - 1)
        def _():
            pltpu.sync_copy(acc.at[pl.ds(base, acc_rows)],
                            out_hbm.at[cid, pl.ds(base, acc_rows)])

        @pl.when(sid == ns - 1)
        def _():
            pltpu.sync_copy(acc.at[pl.ds(base, tail)],
                            out_hbm.at[cid, pl.ds(base, tail)])

    return seg_sum(h, e2)


def _init_matmul(x, w0):
    def body(x_ref, w_ref, o_ref):
        o_ref[...] = lax.dot_general(
            x_ref[...], w_ref[...], (((1,), (1,)), ((), ())),
            preferred_element_type=jnp.float32)

    return pl.pallas_call(
        body,
        out_shape=jax.ShapeDtypeStruct((x.shape[0], w0.shape[0]), jnp.float32),
    )(x, w0)


def _gin_mlp(h, parts, eps, w1, b1, g, be, w2, b2):
    n, d = h.shape

    def body(h_ref, p_ref, eps_ref, w1_ref, b1_ref, g_ref, be_ref, w2_ref,
             b2_ref, o_ref):
        agg = p_ref[0] + p_ref[1]
        t = (1.0 + eps_ref[0, 0]) * h_ref[...] + agg
        u = lax.dot_general(t, w1_ref[...], (((1,), (1,)), ((), ())),
                            preferred_element_type=jnp.float32) + b1_ref[...]
        mean = jnp.mean(u, axis=0, keepdims=True)
        var = jnp.mean(jnp.square(u - mean), axis=0, keepdims=True)
        un = (u - mean) * lax.rsqrt(var + BN_EPS) * g_ref[...] + be_ref[...]
        un = jnp.maximum(un, 0.0)
        v = lax.dot_general(un, w2_ref[...], (((1,), (1,)), ((), ())),
                            preferred_element_type=jnp.float32) + b2_ref[...]
        o_ref[...] = jnp.maximum(v, 0.0)

    return pl.pallas_call(
        body,
        in_specs=[
            pl.BlockSpec(memory_space=pltpu.VMEM),
            pl.BlockSpec(memory_space=pltpu.VMEM),
            pl.BlockSpec(memory_space=pltpu.SMEM),
            pl.BlockSpec(memory_space=pltpu.VMEM),
            pl.BlockSpec(memory_space=pltpu.VMEM),
            pl.BlockSpec(memory_space=pltpu.VMEM),
            pl.BlockSpec(memory_space=pltpu.VMEM),
            pl.BlockSpec(memory_space=pltpu.VMEM),
            pl.BlockSpec(memory_space=pltpu.VMEM),
        ],
        out_shape=jax.ShapeDtypeStruct((n, d), jnp.float32),
    )(h, parts, eps.reshape(1, 1), w1, b1.reshape(1, d), g.reshape(1, d),
      be.reshape(1, d), w2, b2.reshape(1, d))


def kernel(x, edge_index, W0, eps0, W1_0, b1_0, g_0, be_0, W2_0, b2_0,
           eps1, W1_1, b1_1, g_1, be_1, W2_1, b2_1):
    n = x.shape[0]
    d = W0.shape[0]
    e = edge_index.shape[1]
    e2 = edge_index.reshape(2 * e // _C, _C)

    h0 = _init_matmul(x, W0)
    p0 = _seg_sum_partials(h0, e2)
    h1 = _gin_mlp(h0, p0, eps0, W1_0, b1_0, g_0, be_0, W2_0, b2_0)
    p1 = _seg_sum_partials(h1, e2)
    h2 = _gin_mlp(h1, p1, eps1, W1_1, b1_1, g_1, be_1, W2_1, b2_1)
    return h2.reshape(1, n, d)
